# baseline (speedup 1.0000x reference)
import jax
import jax.numpy as jnp
from jax import lax
from jax.experimental import pallas as pl
from jax.experimental.pallas import tpu as pltpu
from jax.experimental.pallas import tpu_sc as plsc

B = 16384
D = 64
NC = 2
NS = 16
NW = NC * NS
BPW = B // NW
CHUNK = 128
NCHUNK = BPW // CHUNK
LANES = 16
NBKT = 256
RING = 4
NROUND = 32


def _worker_base():
    wid = lax.axis_index("s") * NC + lax.axis_index("c")
    return wid, pl.multiple_of(wid * BPW, 128)


def _sread(ref, i):
    return ref[pl.ds(i, LANES)][0]


def _slab_gather_body(tabT, idx_h, rows_out,
                      idx_v, si_v, bp_v, sis_v, bps_v,
                      hist_v, start_v, combo_v, sbkt_v, nt_v, dst_v, rows_v,
                      ring0, ring1, ring2, ring3,
                      sem0, sem1, sem2, sem3, sem_m):
    wid, _ = _worker_base()
    rings = (ring0, ring1, ring2, ring3)
    sems = (sem0, sem1, sem2, sem3)

    pltpu.sync_copy(idx_h, idx_v)

    iota16 = lax.iota(jnp.int32, 16)
    ones16 = jnp.ones((16,), jnp.int32)
    zeros16 = jnp.zeros((16,), jnp.int32)
    e0 = jnp.where(iota16 == 0, 1, 0)
    iotas = [iota16 + (16 * c4) for c4 in range(D // LANES)]

    def round_body(rnd, carry):
        lo = rnd * BPW
        hi = lo + BPW
        run = jnp.logical_or(rnd == 0, lo < _sread(nt_v, 0))

        @pl.when(run)
        def _():
            def fill_bps(j, c):
                bps_v[pl.ds(j * LANES, LANES)] = jnp.full((LANES,), B,
                                                          jnp.int32)
                return c
            lax.fori_loop(0, (BPW + 2 * LANES) // LANES, fill_bps, 0)

            def scan_chunk(i, cf):
                cnt, filled = cf
                iv = idx_v[pl.ds(i * 16, 16)]
                m = (lax.shift_right_logical(iv, 7) & 31) == wid
                mi = jnp.where(m, 1, 0)
                cum = plsc.cumsum(mi)
                rank = cnt + cum - 1
                m2 = jnp.logical_and(m, jnp.logical_and(rank >= lo, rank < hi))
                plsc.store_compressed(si_v.at[pl.ds(filled, 16)], iv, mask=m2)
                plsc.store_compressed(bp_v.at[pl.ds(filled, 16)],
                                      i * 16 + iota16, mask=m2)
                return cnt + cum[15], filled + jnp.sum(jnp.where(m2, 1, 0))

            cnt, nrows = lax.fori_loop(0, B // 16, scan_chunk, (0, 0))
            nt_v[pl.ds(0, LANES)] = jnp.full((LANES,), cnt, jnp.int32)

            @pl.when(nrows > 0)
            def _():
                def zero_hist(j, c):
                    hist_v[pl.ds(j * 16, 16)] = zeros16
                    return c
                lax.fori_loop(0, NBKT // 16, zero_hist, 0)

                def hist_chunk(j, c):
                    iv = si_v[pl.ds(j * 16, 16)]
                    bv = lax.shift_right_logical(iv, 12)
                    m = (j * 16 + iota16) < nrows
                    plsc.addupdate_scatter(hist_v, [bv], ones16, mask=m)
                    return c
                lax.fori_loop(0, (nrows + 15) // 16, hist_chunk, 0)

                def prefix(j, pcar):
                    h = hist_v[pl.ds(j * 16, 16)]
                    inc = plsc.cumsum(h)
                    start_v[pl.ds(j * 16, 16)] = pcar + inc - h
                    return pcar + inc[15]
                lax.fori_loop(0, NBKT // 16, prefix, 0)

                def place(r, c):
                    v = _sread(si_v, r)
                    bpos = _sread(bp_v, r)
                    b = lax.shift_right_logical(v, 12)
                    w16 = start_v[pl.ds(b, 16)]
                    pos = w16[0]
                    start_v[pl.ds(b, 16)] = w16 + e0
                    o1 = sis_v[pl.ds(pos, 16)]
                    sis_v[pl.ds(pos, 16)] = jnp.where(iota16 == 0, v, o1)
                    o2 = bps_v[pl.ds(pos, 16)]
                    bps_v[pl.ds(pos, 16)] = jnp.where(iota16 == 0, bpos, o2)
                    return c
                lax.fori_loop(0, nrows, place, 0)

                def compact(b, nrun):
                    h = _sread(hist_v, b)

                    @pl.when(h > 0)
                    def _():
                        re = _sread(start_v, b)
                        w1 = combo_v[pl.ds(nrun, 16)]
                        combo_v[pl.ds(nrun, 16)] = jnp.where(
                            iota16 == 0, re - h, w1)
                        w2 = sbkt_v[pl.ds(nrun, 16)]
                        sbkt_v[pl.ds(nrun, 16)] = jnp.where(iota16 == 0, b, w2)
                    return nrun + jnp.where(h > 0, 1, 0)
                n = lax.fori_loop(0, NBKT, compact, 0)
                wn = combo_v[pl.ds(n, 16)]
                combo_v[pl.ds(n, 16)] = jnp.where(iota16 == 0, nrows, wn)

                def issue(s, ring_k, sem_k):
                    c = _sread(sbkt_v, s) * 32 + wid
                    cb = pl.multiple_of(c * 128, 128)
                    pltpu.async_copy(tabT.at[:, pl.ds(cb, 128)], ring_k, sem_k)

                for k in range(RING):
                    @pl.when(k < n)
                    def _(k=k):
                        issue(k, rings[k], sems[k])

                def group(gi, gcar):
                    for k in range(RING):
                        s = gi * RING + k
                        ring_k, sem_k = rings[k], sems[k]

                        @pl.when(s < n)
                        def _(s=s, ring_k=ring_k, sem_k=sem_k):
                            pltpu.make_async_copy(tabT.at[:, pl.ds(0, 128)],
                                                  ring_k, sem_k).wait()
                            rs = _sread(combo_v, s)
                            re = _sread(combo_v, s + 1)

                            def rowb(r, cc):
                                lv = jnp.full((16,), _sread(sis_v, r) & 127,
                                              jnp.int32)
                                for c4 in range(D // LANES):
                                    g = plsc.load_gather(ring_k,
                                                         [iotas[c4], lv])
                                    rows_v[r, pl.ds(16 * c4, 16)] = g
                                return cc

                            lax.fori_loop(rs, re, rowb, 0)

                            @pl.when(s + RING < n)
                            def _():
                                issue(s + RING, ring_k, sem_k)
                    return gcar

                lax.fori_loop(0, (NBKT + RING - 1) // RING, group, 0)

                def mkdst(j, c):
                    dst_v[j // 8, pl.ds((j % 8) * 16, 16)] = \
                        bps_v[pl.ds(j * 16, 16)]
                    return c
                lax.fori_loop(0, NCHUNK * 8, mkdst, 0)

                for j in range(NCHUNK):
                    @pl.when(j * CHUNK < nrows)
                    def _(j=j):
                        pltpu.async_copy(rows_v.at[pl.ds(j * CHUNK, CHUNK)],
                                         rows_out.at[dst_v.at[j]], sem_m)
                for j in range(NCHUNK):
                    @pl.when(j * CHUNK < nrows)
                    def _(j=j):
                        pltpu.make_async_copy(
                            rows_v.at[pl.ds(j * CHUNK, CHUNK)],
                            rows_out.at[dst_v.at[0]], sem_m).wait()

        return carry

    lax.fori_loop(0, NROUND, round_body, 0)


def _dot_body(u_rows, p_rows, partials, u_v, p_v, pacc, sem):
    wid, base = _worker_base()
    half = BPW // 2
    acc = jnp.zeros((LANES,), jnp.float32)
    for h in range(2):
        hb = pl.multiple_of(base + h * half, 128)
        cu = pltpu.async_copy(u_rows.at[pl.ds(hb, half)], u_v, sem)
        cp = pltpu.async_copy(p_rows.at[pl.ds(hb, half)], p_v, sem)
        cu.wait()
        cp.wait()

        def dot_chunk(r, a):
            s = a
            for c in range(D // LANES):
                sl = pl.ds(c * LANES, LANES)
                s = s + u_v[r, sl] * p_v[r, sl]
            return s

        acc = lax.fori_loop(0, half, dot_chunk, acc)
    for c in range(8):
        pacc[pl.ds(c * LANES, LANES)] = acc if c == 0 else jnp.zeros(
            (LANES,), jnp.float32)
    pltpu.sync_copy(pacc,
                    partials.at[pl.ds(pl.multiple_of(wid * 128, 128), 128)])


def _combine_body(partials, uidxb, pidxb, ubias, pbias, out,
                  pall, idx_u, idx_p, bu_v, bp_v, ob, sem):
    wid, base = _worker_base()
    pltpu.sync_copy(uidxb.at[pl.ds(wid * NCHUNK, NCHUNK)], idx_u)
    pltpu.sync_copy(pidxb.at[pl.ds(wid * NCHUNK, NCHUNK)], idx_p)
    copies = []
    for j in range(NCHUNK):
        dst = pl.ds(j * CHUNK, CHUNK)
        copies.append(pltpu.async_copy(ubias.at[idx_u.at[j]], bu_v.at[dst], sem))
        copies.append(pltpu.async_copy(pbias.at[idx_p.at[j]], bp_v.at[dst], sem))
    pltpu.sync_copy(partials, pall)
    for c in copies:
        c.wait()

    def sum_body(i, tv):
        return tv + pall[pl.ds(i * 128, LANES)]

    tv = lax.fori_loop(0, NW, sum_body, jnp.zeros((LANES,), jnp.float32))
    total = jnp.sum(tv)

    def sig_body(k, carry):
        sl = pl.ds(k * LANES, LANES)
        x = total + bu_v[sl] + bp_v[sl]
        ob[sl] = 1.0 / (1.0 + jnp.exp(-x))
        return carry

    lax.fori_loop(0, BPW // LANES, sig_body, 0)
    pltpu.sync_copy(ob, out.at[pl.ds(base, BPW)])


def kernel(inputs, user_emb, user_bias, place_emb, place_bias):
    u_idx = inputs[:, 0].astype(jnp.int32)
    p_idx = inputs[:, 1].astype(jnp.int32)
    ub = user_bias.reshape(-1)
    pb = place_bias.reshape(-1)

    def mesh():
        return plsc.VectorSubcoreMesh(core_axis_name="c", subcore_axis_name="s")

    PAD = 32
    slab_fn = pl.kernel(
        _slab_gather_body,
        mesh=mesh(),
        compiler_params=pltpu.CompilerParams(needs_layout_passes=False),
        out_type=jax.ShapeDtypeStruct((B + CHUNK, 128), jnp.float32),
        scratch_types=[
            pltpu.VMEM((B,), jnp.int32),
            pltpu.VMEM((BPW + PAD,), jnp.int32),
            pltpu.VMEM((BPW + PAD,), jnp.int32),
            pltpu.VMEM((BPW + PAD,), jnp.int32),
            pltpu.VMEM((BPW + PAD,), jnp.int32),
            pltpu.VMEM((NBKT + PAD,), jnp.int32),
            pltpu.VMEM((NBKT + PAD,), jnp.int32),
            pltpu.VMEM((NBKT + PAD,), jnp.int32),
            pltpu.VMEM((NBKT + PAD,), jnp.int32),
            pltpu.VMEM((LANES,), jnp.int32),
            pltpu.VMEM((NCHUNK, CHUNK), jnp.int32),
            pltpu.VMEM((BPW, 128), jnp.float32),
            pltpu.VMEM((D, 128), jnp.float32),
            pltpu.VMEM((D, 128), jnp.float32),
            pltpu.VMEM((D, 128), jnp.float32),
            pltpu.VMEM((D, 128), jnp.float32),
            pltpu.SemaphoreType.DMA,
            pltpu.SemaphoreType.DMA,
            pltpu.SemaphoreType.DMA,
            pltpu.SemaphoreType.DMA,
            pltpu.SemaphoreType.DMA,
        ],
    )
    u_rows = slab_fn(user_emb.T, u_idx)
    p_rows = slab_fn(place_emb.T, p_idx)

    dot_fn = pl.kernel(
        _dot_body,
        mesh=mesh(),
        compiler_params=pltpu.CompilerParams(needs_layout_passes=False),
        out_type=jax.ShapeDtypeStruct((NW * 128,), jnp.float32),
        scratch_types=[
            pltpu.VMEM((BPW // 2, 128), jnp.float32),
            pltpu.VMEM((BPW // 2, 128), jnp.float32),
            pltpu.VMEM((128,), jnp.float32),
            pltpu.SemaphoreType.DMA,
        ],
    )
    partials = dot_fn(u_rows, p_rows)

    combine_fn = pl.kernel(
        _combine_body,
        mesh=mesh(),
        compiler_params=pltpu.CompilerParams(
            use_tc_tiling_on_sc=False, needs_layout_passes=False),
        out_type=jax.ShapeDtypeStruct((B,), jnp.float32),
        scratch_types=[
            pltpu.VMEM((NW * 128,), jnp.float32),
            pltpu.VMEM((NCHUNK, CHUNK), jnp.int32),
            pltpu.VMEM((NCHUNK, CHUNK), jnp.int32),
            pltpu.VMEM((BPW,), jnp.float32),
            pltpu.VMEM((BPW,), jnp.float32),
            pltpu.VMEM((BPW,), jnp.float32),
            pltpu.SemaphoreType.DMA,
        ],
    )
    out = combine_fn(partials, u_idx.reshape(B // CHUNK, CHUNK),
                     p_idx.reshape(B // CHUNK, CHUNK), ub, pb)
    return out.reshape(B, 1)


# speedup vs baseline: 1.7886x; 1.7886x over previous
import jax
import jax.numpy as jnp
from jax import lax
from jax.experimental import pallas as pl
from jax.experimental.pallas import tpu as pltpu
from jax.experimental.pallas import tpu_sc as plsc

B = 16384
D = 64
NC = 2
NS = 16
NW = NC * NS
BPW = B // NW
CHUNK = 128
NCHUNK = BPW // CHUNK
LANES = 16
NSLOT = 640
RING = 6


def _worker_base():
    wid = lax.axis_index("s") * NC + lax.axis_index("c")
    return wid, pl.multiple_of(wid * BPW, 128)


def _slab_plan(idx):
    order = jnp.argsort((idx >> 7).astype(jnp.uint16),
                        stable=False).astype(jnp.int32)
    si = jnp.take(idx, order)
    sw = si.reshape(NW, BPW)
    slab = sw >> 7
    first = jnp.concatenate(
        [jnp.ones((NW, 1), bool), slab[:, 1:] != slab[:, :-1]], axis=1)
    r_iota = lax.broadcasted_iota(jnp.int32, (NW, BPW), 1)
    pf = jnp.where(first, r_iota, BPW)
    row_start = jnp.sort(pf, axis=1)
    row_start = jnp.concatenate(
        [row_start, jnp.full((NW, NSLOT - BPW), BPW, jnp.int32)], axis=1)
    nslab = jnp.sum(first.astype(jnp.int32), axis=1).astype(jnp.int32)
    return order, si, row_start.reshape(-1), nslab


def _sread(ref, i):
    return ref[pl.ds(i, LANES)][0]


def _slab_gather_body(tabT, combo_h, si_h, dst_h, n_h, rows_out,
                      combo_v, si_v, n_v, dst_v,
                      rows_v, ring0, ring1, ring2, ring3, ring4, ring5,
                      sem0, sem1, sem2, sem3, sem4, sem5, sem_m):
    wid, base = _worker_base()
    rings = (ring0, ring1, ring2, ring3, ring4, ring5)
    sems = (sem0, sem1, sem2, sem3, sem4, sem5)

    pltpu.sync_copy(combo_h.at[pl.ds(pl.multiple_of(wid * NSLOT, 128), NSLOT)],
                    combo_v.at[pl.ds(0, NSLOT)])
    pltpu.sync_copy(si_h.at[pl.ds(base, BPW)], si_v.at[pl.ds(0, BPW)])
    pltpu.sync_copy(n_h, n_v.at[pl.ds(0, NW)])
    for j in range(NCHUNK):
        pltpu.sync_copy(dst_h.at[pl.ds(base + j * CHUNK, CHUNK)], dst_v.at[j])
    n = _sread(n_v, wid)

    def issue(s, ring_k, sem_k):
        c = lax.shift_right_logical(_sread(si_v, _sread(combo_v, s)), 7)
        cb = pl.multiple_of(c * 128, 128)
        pltpu.async_copy(tabT.at[:, pl.ds(cb, 128)], ring_k, sem_k)

    for k in range(RING):
        @pl.when(k < n)
        def _(k=k):
            issue(k, rings[k], sems[k])

    iotas = [lax.iota(jnp.int32, 16) + (16 * c4) for c4 in range(D // LANES)]

    def group(gi, carry):
        for k in range(RING):
            s = gi * RING + k
            ring_k, sem_k = rings[k], sems[k]

            @pl.when(s < n)
            def _(s=s, ring_k=ring_k, sem_k=sem_k):
                pltpu.make_async_copy(tabT.at[:, pl.ds(0, 128)],
                                      ring_k, sem_k).wait()
                rs = _sread(combo_v, s)
                re = _sread(combo_v, s + 1)

                def rowb(r, cc):
                    lv = jnp.full((16,), _sread(si_v, r) & 127, jnp.int32)
                    for c4 in range(D // LANES):
                        g = plsc.load_gather(ring_k, [iotas[c4], lv])
                        rows_v[r, pl.ds(16 * c4, 16)] = g
                    return cc

                lax.fori_loop(rs, re, rowb, 0)

                @pl.when(s + RING < n)
                def _():
                    issue(s + RING, ring_k, sem_k)
        return carry

    lax.fori_loop(0, NSLOT // RING, group, 0)

    for j in range(NCHUNK):
        pltpu.async_copy(rows_v.at[pl.ds(j * CHUNK, CHUNK)],
                         rows_out.at[dst_v.at[j]], sem_m)
    for j in range(NCHUNK):
        pltpu.make_async_copy(rows_v.at[pl.ds(j * CHUNK, CHUNK)],
                              rows_out.at[dst_v.at[0]], sem_m).wait()


def _dot_body(u_rows, p_rows, partials, u_v, p_v, pacc, sem):
    wid, base = _worker_base()
    half = BPW // 2
    acc = jnp.zeros((LANES,), jnp.float32)
    for h in range(2):
        hb = pl.multiple_of(base + h * half, 128)
        cu = pltpu.async_copy(u_rows.at[pl.ds(hb, half)], u_v, sem)
        cp = pltpu.async_copy(p_rows.at[pl.ds(hb, half)], p_v, sem)
        cu.wait()
        cp.wait()

        def dot_chunk(r, a):
            s = a
            for c in range(D // LANES):
                sl = pl.ds(c * LANES, LANES)
                s = s + u_v[r, sl] * p_v[r, sl]
            return s

        acc = lax.fori_loop(0, half, dot_chunk, acc)
    for c in range(8):
        pacc[pl.ds(c * LANES, LANES)] = acc if c == 0 else jnp.zeros(
            (LANES,), jnp.float32)
    pltpu.sync_copy(pacc, partials.at[pl.ds(pl.multiple_of(wid * 128, 128), 128)])


def _combine_body(partials, uidxb, pidxb, ubias, pbias, out,
                  pall, idx_u, idx_p, bu_v, bp_v, ob, sem):
    wid, base = _worker_base()
    pltpu.sync_copy(uidxb.at[pl.ds(wid * NCHUNK, NCHUNK)], idx_u)
    pltpu.sync_copy(pidxb.at[pl.ds(wid * NCHUNK, NCHUNK)], idx_p)
    copies = []
    for j in range(NCHUNK):
        dst = pl.ds(j * CHUNK, CHUNK)
        copies.append(pltpu.async_copy(ubias.at[idx_u.at[j]], bu_v.at[dst], sem))
        copies.append(pltpu.async_copy(pbias.at[idx_p.at[j]], bp_v.at[dst], sem))
    pltpu.sync_copy(partials, pall)
    for c in copies:
        c.wait()

    def sum_body(i, tv):
        return tv + pall[pl.ds(i * 128, LANES)]

    tv = lax.fori_loop(0, NW, sum_body, jnp.zeros((LANES,), jnp.float32))
    total = jnp.sum(tv)

    def sig_body(k, carry):
        sl = pl.ds(k * LANES, LANES)
        x = total + bu_v[sl] + bp_v[sl]
        ob[sl] = 1.0 / (1.0 + jnp.exp(-x))
        return carry

    lax.fori_loop(0, BPW // LANES, sig_body, 0)
    pltpu.sync_copy(ob, out.at[pl.ds(base, BPW)])


def kernel(inputs, user_emb, user_bias, place_emb, place_bias):
    u_idx = inputs[:, 0].astype(jnp.int32)
    p_idx = inputs[:, 1].astype(jnp.int32)
    ub = user_bias.reshape(-1)
    pb = place_bias.reshape(-1)

    def mesh():
        return plsc.VectorSubcoreMesh(core_axis_name="c", subcore_axis_name="s")

    slab_fn = pl.kernel(
        _slab_gather_body,
        mesh=mesh(),
        compiler_params=pltpu.CompilerParams(needs_layout_passes=False),
        out_type=jax.ShapeDtypeStruct((B, 128), jnp.float32),
        scratch_types=[
            pltpu.VMEM((NSLOT + LANES,), jnp.int32),
            pltpu.VMEM((BPW + LANES,), jnp.int32),
            pltpu.VMEM((NW + LANES,), jnp.int32),
            pltpu.VMEM((NCHUNK, CHUNK), jnp.int32),
            pltpu.VMEM((BPW, 128), jnp.float32),
            pltpu.VMEM((D, 128), jnp.float32),
            pltpu.VMEM((D, 128), jnp.float32),
            pltpu.VMEM((D, 128), jnp.float32),
            pltpu.VMEM((D, 128), jnp.float32),
            pltpu.VMEM((D, 128), jnp.float32),
            pltpu.VMEM((D, 128), jnp.float32),
            pltpu.SemaphoreType.DMA,
            pltpu.SemaphoreType.DMA,
            pltpu.SemaphoreType.DMA,
            pltpu.SemaphoreType.DMA,
            pltpu.SemaphoreType.DMA,
            pltpu.SemaphoreType.DMA,
            pltpu.SemaphoreType.DMA,
        ],
    )
    order_u, si_u, combo_u, n_u = _slab_plan(u_idx)
    order_p, si_p, combo_p, n_p = _slab_plan(p_idx)
    u_rows = slab_fn(user_emb.T, combo_u, si_u, order_u, n_u)
    p_rows = slab_fn(place_emb.T, combo_p, si_p, order_p, n_p)

    dot_fn = pl.kernel(
        _dot_body,
        mesh=mesh(),
        compiler_params=pltpu.CompilerParams(needs_layout_passes=False),
        out_type=jax.ShapeDtypeStruct((NW * 128,), jnp.float32),
        scratch_types=[
            pltpu.VMEM((BPW // 2, 128), jnp.float32),
            pltpu.VMEM((BPW // 2, 128), jnp.float32),
            pltpu.VMEM((128,), jnp.float32),
            pltpu.SemaphoreType.DMA,
        ],
    )
    partials = dot_fn(u_rows, p_rows)

    combine_fn = pl.kernel(
        _combine_body,
        mesh=mesh(),
        compiler_params=pltpu.CompilerParams(
            use_tc_tiling_on_sc=False, needs_layout_passes=False),
        out_type=jax.ShapeDtypeStruct((B,), jnp.float32),
        scratch_types=[
            pltpu.VMEM((NW * 128,), jnp.float32),
            pltpu.VMEM((NCHUNK, CHUNK), jnp.int32),
            pltpu.VMEM((NCHUNK, CHUNK), jnp.int32),
            pltpu.VMEM((BPW,), jnp.float32),
            pltpu.VMEM((BPW,), jnp.float32),
            pltpu.VMEM((BPW,), jnp.float32),
            pltpu.SemaphoreType.DMA,
        ],
    )
    out = combine_fn(partials, u_idx.reshape(B // CHUNK, CHUNK),
                     p_idx.reshape(B // CHUNK, CHUNK), ub, pb)
    return out.reshape(B, 1)
